# baseline (speedup 1.0000x reference)
import functools

import jax
import jax.numpy as jnp
from jax import lax
from jax.experimental import pallas as pl
from jax.experimental.pallas import tpu as pltpu
from jax.experimental.pallas import tpu_sc as plsc

NC = 2
NS = 16
NW = NC * NS

_SC_PARAMS = pltpu.CompilerParams(use_tc_tiling_on_sc=False)


def _sc_segsum(x2, src2, dst2, n, dh, with_count=False):
  _, nchunk, cc = src2.shape
  nacc = n + 128
  rpw = n // NS
  zr = 128
  while rpw % zr:
    zr -= 1
  nbuf = 1
  for nb in (5, 4, 3, 2):
    if nchunk % nb == 0:
      nbuf = nb
      break
  mesh = plsc.VectorSubcoreMesh(core_axis_name="c", subcore_axis_name="s")

  out_type = [jax.ShapeDtypeStruct((NC, n, dh), jnp.float32)]
  cnt_scratch = []
  if with_count:
    out_type.append(jax.ShapeDtypeStruct((NC, n, 16), jnp.float32))
    cnt_scratch = [
        pltpu.VMEM((cc, 16), jnp.float32),
        pltpu.VMEM((zr, 16), jnp.float32),
        pltpu.VMEM_SHARED((nacc, 16), jnp.float32),
    ]

  @functools.partial(
      pl.kernel,
      out_type=out_type,
      mesh=mesh,
      compiler_params=_SC_PARAMS,
      scratch_types=[
          pltpu.VMEM((nchunk, cc), jnp.int32),
          [pltpu.VMEM((cc,), jnp.int32) for _ in range(nbuf)],
          [pltpu.VMEM((cc,), jnp.int32) for _ in range(nbuf)],
          [pltpu.VMEM((cc, dh), jnp.float32) for _ in range(nbuf)],
          pltpu.VMEM((zr, dh), jnp.float32),
          pltpu.VMEM_SHARED((nacc, dh), jnp.float32),
          [pltpu.SemaphoreType.DMA for _ in range(nbuf)],
          [pltpu.SemaphoreType.DMA for _ in range(nbuf)],
      ] + cnt_scratch,
  )
  def seg(x2_hbm, src_hbm, dst_hbm, out_hbm, *rest):
    if with_count:
      cnt_hbm, sidx, idx2, dcur, rows, zbuf, agg, gsem, dsem, \
          ones, zcnt, cnt = rest
    else:
      (sidx, idx2, dcur, rows, zbuf, agg, gsem, dsem) = rest
    c = lax.axis_index("c")
    s = lax.axis_index("s")

    @pl.loop(0, zr)
    def _(r):
      @pl.loop(0, dh // 16)
      def _(j):
        zbuf[r, pl.ds(j * 16, 16)] = jnp.zeros((16,), jnp.float32)

    @pl.loop(0, rpw // zr)
    def _(t):
      pltpu.sync_copy(zbuf, agg.at[pl.ds(s * rpw + t * zr, zr)])

    if with_count:
      @pl.loop(0, cc)
      def _(r):
        ones[r, :] = jnp.ones((16,), jnp.float32)

      @pl.loop(0, zr)
      def _(r):
        zcnt[r, :] = jnp.zeros((16,), jnp.float32)

      @pl.loop(0, rpw // zr)
      def _(t):
        pltpu.sync_copy(zcnt, cnt.at[pl.ds(s * rpw + t * zr, zr)])

    pltpu.sync_copy(src_hbm.at[s], sidx)

    def fire(k, b):
      pltpu.async_copy(dst_hbm.at[s, k], dcur[b], dsem[b])
      for j in range(cc // 16):
        v = sidx[k, pl.ds(j * 16, 16)]
        idx2[b][pl.ds(j * 16, 16)] = v * 2 + c
      pltpu.async_copy(x2_hbm.at[idx2[b]], rows[b], gsem[b])

    plsc.subcore_barrier()

    for b in range(nbuf):
      fire(b, b)

    @pl.loop(0, nchunk, step=nbuf)
    def _(g):
      for b in range(nbuf):
        pltpu.make_async_copy(dst_hbm.at[s, 0], dcur[b], dsem[b]).wait()
        pltpu.make_async_copy(x2_hbm.at[idx2[b]], rows[b], gsem[b]).wait()
        pltpu.sync_copy(rows[b], agg.at[dcur[b]], add=True)
        if with_count:
          @pl.when(((g + b) % 2) == c)
          def _():
            pltpu.sync_copy(ones, cnt.at[dcur[b]], add=True)

        @pl.when(g < nchunk - nbuf)
        def _():
          fire(g + nbuf + b, b)

    plsc.subcore_barrier()
    pltpu.sync_copy(agg.at[pl.ds(s * rpw, rpw)],
                    out_hbm.at[c, pl.ds(s * rpw, rpw)])
    if with_count:
      pltpu.sync_copy(cnt.at[pl.ds(s * rpw, rpw)],
                      cnt_hbm.at[c, pl.ds(s * rpw, rpw)])

  res = seg(x2, src2, dst2)
  return res if with_count else res[0]


def _dotT(a, w):
  return lax.dot_general(a, w, (((1,), (1,)), ((), ())),
                         preferred_element_type=jnp.float32)


def _tc_dense1(p, cntp, xin, wl, bl, wr, blk):
  n, d = xin.shape
  dh = d // NC
  grid = (n // blk,)

  def body(p_ref, cnt_ref, x_ref, wl_ref, bl_ref, wr_ref, h_ref, ci_ref):
    cnt = cnt_ref[0, :, :1] + cnt_ref[1, :, :1]
    cinv = 1.0 / jnp.maximum(cnt, 1.0)
    ci_ref[...] = cinv
    agg = jnp.concatenate([p_ref[0], p_ref[1]], axis=1) * cinv
    h = _dotT(agg, wl_ref[...]) + _dotT(x_ref[...], wr_ref[...]) + bl_ref[...]
    h_ref[...] = jnp.maximum(h, 0.0)

  return pl.pallas_call(
      body,
      grid=grid,
      in_specs=[
          pl.BlockSpec((NC, blk, dh), lambda i: (0, i, 0)),
          pl.BlockSpec((NC, blk, 16), lambda i: (0, i, 0)),
          pl.BlockSpec((blk, d), lambda i: (i, 0)),
          pl.BlockSpec((d, d), lambda i: (0, 0)),
          pl.BlockSpec((d,), lambda i: (0,)),
          pl.BlockSpec((d, d), lambda i: (0, 0)),
      ],
      out_specs=[
          pl.BlockSpec((blk, d), lambda i: (i, 0)),
          pl.BlockSpec((blk, 1), lambda i: (i, 0)),
      ],
      out_shape=[
          jax.ShapeDtypeStruct((n, d), jnp.float32),
          jax.ShapeDtypeStruct((n, 1), jnp.float32),
      ],
  )(p, cntp, xin, wl, bl, wr)


def _tc_dense(p, cinv, xin, wl, bl, wr, blk):
  n, d = xin.shape
  dh = d // NC
  grid = (n // blk,)

  def body(p_ref, ci_ref, x_ref, wl_ref, bl_ref, wr_ref, h_ref):
    agg = jnp.concatenate([p_ref[0], p_ref[1]], axis=1) * ci_ref[...]
    h = _dotT(agg, wl_ref[...]) + _dotT(x_ref[...], wr_ref[...]) + bl_ref[...]
    h_ref[...] = jnp.maximum(h, 0.0)

  return pl.pallas_call(
      body,
      grid=grid,
      in_specs=[
          pl.BlockSpec((NC, blk, dh), lambda i: (0, i, 0)),
          pl.BlockSpec((blk, 1), lambda i: (i, 0)),
          pl.BlockSpec((blk, d), lambda i: (i, 0)),
          pl.BlockSpec((d, d), lambda i: (0, 0)),
          pl.BlockSpec((d,), lambda i: (0,)),
          pl.BlockSpec((d, d), lambda i: (0, 0)),
      ],
      out_specs=pl.BlockSpec((blk, d), lambda i: (i, 0)),
      out_shape=jax.ShapeDtypeStruct((n, d), jnp.float32),
  )(p, cinv, xin, wl, bl, wr)


def _tc_dense3(p, cinv, xin, wl, bl, wr, w1, b1, w2, b2, blk):
  n, d = xin.shape
  dh = d // NC
  dout = w2.shape[0]
  grid = (n // blk,)

  def body(p_ref, ci_ref, x_ref, wl_ref, bl_ref, wr_ref,
           w1_ref, b1_ref, w2_ref, b2_ref, o_ref):
    agg = jnp.concatenate([p_ref[0], p_ref[1]], axis=1) * ci_ref[...]
    h = _dotT(agg, wl_ref[...]) + _dotT(x_ref[...], wr_ref[...]) + bl_ref[...]
    h = jnp.maximum(h, 0.0)
    t = jnp.maximum(_dotT(h, w1_ref[...]) + b1_ref[...], 0.0)
    o_ref[...] = _dotT(t, w2_ref[...]) + b2_ref[...]

  return pl.pallas_call(
      body,
      grid=grid,
      in_specs=[
          pl.BlockSpec((NC, blk, dh), lambda i: (0, i, 0)),
          pl.BlockSpec((blk, 1), lambda i: (i, 0)),
          pl.BlockSpec((blk, d), lambda i: (i, 0)),
          pl.BlockSpec((d, d), lambda i: (0, 0)),
          pl.BlockSpec((d,), lambda i: (0,)),
          pl.BlockSpec((d, d), lambda i: (0, 0)),
          pl.BlockSpec((d, d), lambda i: (0, 0)),
          pl.BlockSpec((d,), lambda i: (0,)),
          pl.BlockSpec((dout, d), lambda i: (0, 0)),
          pl.BlockSpec((dout,), lambda i: (0,)),
      ],
      out_specs=pl.BlockSpec((blk, dout), lambda i: (i, 0)),
      out_shape=jax.ShapeDtypeStruct((n, dout), jnp.float32),
  )(p, cinv, xin, wl, bl, wr, w1, b1, w2, b2)


def kernel(x, edge_index, Wl1, bl1, Wr1, Wl2, bl2, Wr2, Wl3, bl3, Wr3,
           W_lin1, b_lin1, W_lin2, b_lin2):
  n, d = x.shape
  dh = d // NC
  src = edge_index[0]
  dst = edge_index[1]
  blk = 1000

  e = src.shape[0]
  cc = 128
  eps = -(-e // (NS * cc)) * cc
  nchunk = eps // cc
  pad = NS * eps - e
  if pad:
    src = jnp.concatenate([src, jnp.zeros((pad,), jnp.int32)])
    dst = jnp.concatenate([dst, n + (jnp.arange(pad, dtype=jnp.int32) % 128)])
  src2 = src.reshape(NS, nchunk, cc)
  dst2 = dst.reshape(NS, nchunk, cc)

  p1, cntp = _sc_segsum(x.reshape(NC * n, dh), src2, dst2, n, dh,
                        with_count=True)
  h1, cinv = _tc_dense1(p1, cntp, x, Wl1, bl1, Wr1, blk)
  p2 = _sc_segsum(h1.reshape(NC * n, dh), src2, dst2, n, dh)
  h2 = _tc_dense(p2, cinv, h1, Wl2, bl2, Wr2, blk)
  p3 = _sc_segsum(h2.reshape(NC * n, dh), src2, dst2, n, dh)
  out = _tc_dense3(p3, cinv, h2, Wl3, bl3, Wr3,
                   W_lin1, b_lin1, W_lin2, b_lin2, blk)
  return out


# speedup vs baseline: 2.3943x; 2.3943x over previous
import functools

import jax
import jax.numpy as jnp
from jax import lax
from jax.experimental import pallas as pl
from jax.experimental.pallas import tpu as pltpu
from jax.experimental.pallas import tpu_sc as plsc

NC = 2
NS = 16
NW = NC * NS

_SC_PARAMS = pltpu.CompilerParams(use_tc_tiling_on_sc=False)


def _sc_segsum(x2, src2, dst2, n, dh, with_count=False):
  _, nchunk, cc = src2.shape
  nacc = n + 128
  rpw = n // NS
  zr = 128
  while rpw % zr:
    zr -= 1
  nbuf = 1
  for nb in (5, 4, 3, 2):
    if nchunk % nb == 0:
      nbuf = nb
      break
  mesh = plsc.VectorSubcoreMesh(core_axis_name="c", subcore_axis_name="s")

  out_type = [jax.ShapeDtypeStruct((NC, n, dh), jnp.float32)]
  cnt_scratch = []
  if with_count:
    out_type.append(jax.ShapeDtypeStruct((NC, n, 16), jnp.float32))
    cnt_scratch = [
        pltpu.VMEM((cc, 16), jnp.float32),
        pltpu.VMEM((zr, 16), jnp.float32),
        pltpu.VMEM_SHARED((nacc, 16), jnp.float32),
    ]

  @functools.partial(
      pl.kernel,
      out_type=out_type,
      mesh=mesh,
      compiler_params=_SC_PARAMS,
      scratch_types=[
          pltpu.VMEM((nchunk, cc), jnp.int32),
          [pltpu.VMEM((cc,), jnp.int32) for _ in range(nbuf)],
          [pltpu.VMEM((cc,), jnp.int32) for _ in range(nbuf)],
          [pltpu.VMEM((cc, dh), jnp.float32) for _ in range(nbuf)],
          pltpu.VMEM((zr, dh), jnp.float32),
          pltpu.VMEM_SHARED((nacc, dh), jnp.float32),
          [pltpu.SemaphoreType.DMA for _ in range(nbuf)],
          [pltpu.SemaphoreType.DMA for _ in range(nbuf)],
      ] + cnt_scratch,
  )
  def seg(x2_hbm, src_hbm, dst_hbm, out_hbm, *rest):
    if with_count:
      cnt_hbm, sidx, idx2, dcur, rows, zbuf, agg, gsem, dsem, \
          ones, zcnt, cnt = rest
    else:
      (sidx, idx2, dcur, rows, zbuf, agg, gsem, dsem) = rest
    c = lax.axis_index("c")
    s = lax.axis_index("s")

    @pl.loop(0, zr)
    def _(r):
      @pl.loop(0, dh // 16)
      def _(j):
        zbuf[r, pl.ds(j * 16, 16)] = jnp.zeros((16,), jnp.float32)

    @pl.loop(0, rpw // zr)
    def _(t):
      pltpu.sync_copy(zbuf, agg.at[pl.ds(s * rpw + t * zr, zr)])

    if with_count:
      @pl.loop(0, cc)
      def _(r):
        ones[r, :] = jnp.ones((16,), jnp.float32)

      @pl.loop(0, zr)
      def _(r):
        zcnt[r, :] = jnp.zeros((16,), jnp.float32)

      @pl.loop(0, rpw // zr)
      def _(t):
        pltpu.sync_copy(zcnt, cnt.at[pl.ds(s * rpw + t * zr, zr)])

    pltpu.sync_copy(src_hbm.at[s], sidx)

    def fire(k, b):
      pltpu.async_copy(dst_hbm.at[s, k], dcur[b], dsem[b])
      for j in range(cc // 16):
        v = sidx[k, pl.ds(j * 16, 16)]
        idx2[b][pl.ds(j * 16, 16)] = v * 2 + c
      pltpu.async_copy(x2_hbm.at[idx2[b]], rows[b], gsem[b])

    plsc.subcore_barrier()

    for b in range(nbuf):
      fire(b, b)

    @pl.loop(0, nchunk, step=nbuf)
    def _(g):
      for b in range(nbuf):
        pltpu.make_async_copy(dst_hbm.at[s, 0], dcur[b], dsem[b]).wait()
        pltpu.make_async_copy(x2_hbm.at[idx2[b]], rows[b], gsem[b]).wait()
        pltpu.sync_copy(rows[b], agg.at[dcur[b]], add=True)
        if with_count:
          @pl.when(((g + b) % 2) == c)
          def _():
            pltpu.sync_copy(ones, cnt.at[dcur[b]], add=True)

        @pl.when(g < nchunk - nbuf)
        def _():
          fire(g + nbuf + b, b)

    plsc.subcore_barrier()
    pltpu.sync_copy(agg.at[pl.ds(s * rpw, rpw)],
                    out_hbm.at[c, pl.ds(s * rpw, rpw)])
    if with_count:
      pltpu.sync_copy(cnt.at[pl.ds(s * rpw, rpw)],
                      cnt_hbm.at[c, pl.ds(s * rpw, rpw)])

  res = seg(x2, src2, dst2)
  return res if with_count else res[0]


def _dotT(a, w):
  return lax.dot_general(a, w, (((1,), (1,)), ((), ())),
                         preferred_element_type=jnp.float32)


def _tc_dense1(p, cntp, xin, wl, bl, wr, blk):
  n, d = xin.shape
  dh = d // NC
  grid = (n // blk,)

  def body(p_ref, cnt_ref, x_ref, wl_ref, bl_ref, wr_ref, h_ref, ci_ref):
    cnt = cnt_ref[0, :, :1] + cnt_ref[1, :, :1]
    cinv = 1.0 / jnp.maximum(cnt, 1.0)
    ci_ref[...] = cinv
    agg = jnp.concatenate([p_ref[0], p_ref[1]], axis=1) * cinv
    h = _dotT(agg, wl_ref[...]) + _dotT(x_ref[...], wr_ref[...]) + bl_ref[...]
    h_ref[...] = jnp.maximum(h, 0.0)

  return pl.pallas_call(
      body,
      grid=grid,
      in_specs=[
          pl.BlockSpec((NC, blk, dh), lambda i: (0, i, 0)),
          pl.BlockSpec((NC, blk, 16), lambda i: (0, i, 0)),
          pl.BlockSpec((blk, d), lambda i: (i, 0)),
          pl.BlockSpec((d, d), lambda i: (0, 0)),
          pl.BlockSpec((d,), lambda i: (0,)),
          pl.BlockSpec((d, d), lambda i: (0, 0)),
      ],
      out_specs=[
          pl.BlockSpec((blk, d), lambda i: (i, 0)),
          pl.BlockSpec((blk, 1), lambda i: (i, 0)),
      ],
      out_shape=[
          jax.ShapeDtypeStruct((n, d), jnp.float32),
          jax.ShapeDtypeStruct((n, 1), jnp.float32),
      ],
  )(p, cntp, xin, wl, bl, wr)


def _tc_dense(p, cinv, xin, wl, bl, wr, blk):
  n, d = xin.shape
  dh = d // NC
  grid = (n // blk,)

  def body(p_ref, ci_ref, x_ref, wl_ref, bl_ref, wr_ref, h_ref):
    agg = jnp.concatenate([p_ref[0], p_ref[1]], axis=1) * ci_ref[...]
    h = _dotT(agg, wl_ref[...]) + _dotT(x_ref[...], wr_ref[...]) + bl_ref[...]
    h_ref[...] = jnp.maximum(h, 0.0)

  return pl.pallas_call(
      body,
      grid=grid,
      in_specs=[
          pl.BlockSpec((NC, blk, dh), lambda i: (0, i, 0)),
          pl.BlockSpec((blk, 1), lambda i: (i, 0)),
          pl.BlockSpec((blk, d), lambda i: (i, 0)),
          pl.BlockSpec((d, d), lambda i: (0, 0)),
          pl.BlockSpec((d,), lambda i: (0,)),
          pl.BlockSpec((d, d), lambda i: (0, 0)),
      ],
      out_specs=pl.BlockSpec((blk, d), lambda i: (i, 0)),
      out_shape=jax.ShapeDtypeStruct((n, d), jnp.float32),
  )(p, cinv, xin, wl, bl, wr)


def _tc_dense3(p, cinv, xin, wl, bl, wr, w1, b1, w2, b2, blk):
  n, d = xin.shape
  dh = d // NC
  dout = w2.shape[0]
  grid = (n // blk,)

  def body(p_ref, ci_ref, x_ref, wl_ref, bl_ref, wr_ref,
           w1_ref, b1_ref, w2_ref, b2_ref, o_ref):
    agg = jnp.concatenate([p_ref[0], p_ref[1]], axis=1) * ci_ref[...]
    h = _dotT(agg, wl_ref[...]) + _dotT(x_ref[...], wr_ref[...]) + bl_ref[...]
    h = jnp.maximum(h, 0.0)
    t = jnp.maximum(_dotT(h, w1_ref[...]) + b1_ref[...], 0.0)
    o_ref[...] = _dotT(t, w2_ref[...]) + b2_ref[...]

  return pl.pallas_call(
      body,
      grid=grid,
      in_specs=[
          pl.BlockSpec((NC, blk, dh), lambda i: (0, i, 0)),
          pl.BlockSpec((blk, 1), lambda i: (i, 0)),
          pl.BlockSpec((blk, d), lambda i: (i, 0)),
          pl.BlockSpec((d, d), lambda i: (0, 0)),
          pl.BlockSpec((d,), lambda i: (0,)),
          pl.BlockSpec((d, d), lambda i: (0, 0)),
          pl.BlockSpec((d, d), lambda i: (0, 0)),
          pl.BlockSpec((d,), lambda i: (0,)),
          pl.BlockSpec((dout, d), lambda i: (0, 0)),
          pl.BlockSpec((dout,), lambda i: (0,)),
      ],
      out_specs=pl.BlockSpec((blk, dout), lambda i: (i, 0)),
      out_shape=jax.ShapeDtypeStruct((n, dout), jnp.float32),
  )(p, cinv, xin, wl, bl, wr, w1, b1, w2, b2)


def kernel(x, edge_index, Wl1, bl1, Wr1, Wl2, bl2, Wr2, Wl3, bl3, Wr3,
           W_lin1, b_lin1, W_lin2, b_lin2):
  n, d = x.shape
  dh = d // NC
  src = edge_index[0]
  dst = edge_index[1]
  blk = 1000

  e = src.shape[0]
  cc = 80
  eps = -(-e // (NS * cc)) * cc
  nchunk = eps // cc
  pad = NS * eps - e
  if pad:
    src = jnp.concatenate([src, jnp.zeros((pad,), jnp.int32)])
    dst = jnp.concatenate([dst, n + (jnp.arange(pad, dtype=jnp.int32) % 128)])
  src2 = src.reshape(NS, nchunk, cc)
  dst2 = dst.reshape(NS, nchunk, cc)

  p1, cntp = _sc_segsum(x.reshape(NC * n, dh), src2, dst2, n, dh,
                        with_count=True)
  h1, cinv = _tc_dense1(p1, cntp, x, Wl1, bl1, Wr1, blk)
  p2 = _sc_segsum(h1.reshape(NC * n, dh), src2, dst2, n, dh)
  h2 = _tc_dense(p2, cinv, h1, Wl2, bl2, Wr2, blk)
  p3 = _sc_segsum(h2.reshape(NC * n, dh), src2, dst2, n, dh)
  out = _tc_dense3(p3, cinv, h2, Wl3, bl3, Wr3,
                   W_lin1, b_lin1, W_lin2, b_lin2, blk)
  return out
